# Initial kernel scaffold; baseline (speedup 1.0000x reference)
import jax
import jax.numpy as jnp
from jax.experimental import pallas as pl


def kernel(hidden_states, position_ids, ln1_g, ln1_b, ln2_g, ln2_b, wq, wk, wv, wkc, wvc, wqa, wqg, wov, wo, gate_w, ew1, ew2, ew3, sw1, sw2, sw3):
    raise NotImplementedError("write your pallas kernel here")


# speedup vs baseline: 1.2086x; 1.2086x over previous
import functools

import jax
import jax.numpy as jnp
from jax import lax
from jax.experimental import pallas as pl
from jax.experimental.pallas import tpu as pltpu
from jax.experimental.pallas import tpu_sc as plsc

S = 2048
D = 2048
H = 16
HD = 128
ROPE = 64
KVL = 64
FF = 4096
E = 8
TOPK = 2

BM = 256
P = S * TOPK + E * BM
NBLK = P // BM
GE_BN = 512
NJ_E = FF // GE_BN

SH_BN = 512
NJ_S = FF // SH_BN


def _ln_body(x_ref, g_ref, b_ref, o_ref):
    x = x_ref[...]
    m = jnp.mean(x, axis=-1, keepdims=True)
    v = jnp.mean((x - m) ** 2, axis=-1, keepdims=True)
    o_ref[...] = (x - m) * jax.lax.rsqrt(v + 1e-5) * g_ref[...] + b_ref[...]


def _layernorm(x, g, b):
    bm = 256
    return pl.pallas_call(
        _ln_body,
        grid=(S // bm,),
        in_specs=[
            pl.BlockSpec((bm, D), lambda i: (i, 0)),
            pl.BlockSpec((1, D), lambda i: (0, 0)),
            pl.BlockSpec((1, D), lambda i: (0, 0)),
        ],
        out_specs=pl.BlockSpec((bm, D), lambda i: (i, 0)),
        out_shape=jax.ShapeDtypeStruct((S, D), jnp.float32),
    )(x, g.reshape(1, D), b.reshape(1, D))


def _mm_body(x_ref, w_ref, o_ref):
    o_ref[...] = jnp.dot(x_ref[...], w_ref[...],
                         preferred_element_type=jnp.float32)


def _matmul(x, w):
    m, k = x.shape
    _, n = w.shape
    bm, bn = 256, 512
    return pl.pallas_call(
        _mm_body,
        grid=(m // bm, n // bn),
        in_specs=[
            pl.BlockSpec((bm, k), lambda i, j: (i, 0)),
            pl.BlockSpec((k, bn), lambda i, j: (0, j)),
        ],
        out_specs=pl.BlockSpec((bm, bn), lambda i, j: (i, j)),
        out_shape=jax.ShapeDtypeStruct((m, n), jnp.float32),
    )(x, w)


def _mm_add_body(x_ref, w_ref, r_ref, o_ref):
    o_ref[...] = r_ref[...] + jnp.dot(x_ref[...], w_ref[...],
                                      preferred_element_type=jnp.float32)


def _matmul_add(x, w, res):
    m, k = x.shape
    _, n = w.shape
    bm, bn = 256, 512
    return pl.pallas_call(
        _mm_add_body,
        grid=(m // bm, n // bn),
        in_specs=[
            pl.BlockSpec((bm, k), lambda i, j: (i, 0)),
            pl.BlockSpec((k, bn), lambda i, j: (0, j)),
            pl.BlockSpec((bm, bn), lambda i, j: (i, j)),
        ],
        out_specs=pl.BlockSpec((bm, bn), lambda i, j: (i, j)),
        out_shape=jax.ShapeDtypeStruct((m, n), jnp.float32),
    )(x, w, res)


def _rot_half(x):
    h = x.shape[-1] // 2
    return jnp.concatenate((-x[..., h:], x[..., :h]), axis=-1)


def _attn_body(q_ref, k_ref, v_ref, cos_ref, sin_ref, wkc_ref, wvc_ref,
               wqa_ref, wqg_ref, wov_ref, o_ref):
    cos = cos_ref[...]
    sin = sin_ref[...]
    q = q_ref[...]
    k = k_ref[...]
    qr = q[:, :ROPE] * cos + _rot_half(q[:, :ROPE]) * sin
    kr = k[:, :ROPE] * cos + _rot_half(k[:, :ROPE]) * sin
    qf = jnp.concatenate([qr, q[:, ROPE:]], axis=-1)
    kf = jnp.concatenate([kr, k[:, ROPE:]], axis=-1)
    k_c = jnp.dot(kf, wkc_ref[...], preferred_element_type=jnp.float32)
    v_c = jnp.dot(v_ref[...], wvc_ref[...], preferred_element_type=jnp.float32)
    q_a = jnp.dot(qf, wqa_ref[...], preferred_element_type=jnp.float32)
    q_g = jnp.dot(qf, wqg_ref[...], preferred_element_type=jnp.float32)
    s = lax.dot_general(q_a, k_c, (((1,), (1,)), ((), ())),
                        preferred_element_type=jnp.float32)
    s = s * (1.0 / (KVL ** 0.5))
    s = s - jnp.max(s, axis=-1, keepdims=True)
    p = jnp.exp(s)
    p = p / jnp.sum(p, axis=-1, keepdims=True)
    o_c = jnp.dot(p, v_c, preferred_element_type=jnp.float32)
    up = jnp.dot(o_c, wov_ref[...], preferred_element_type=jnp.float32)
    o_ref[...] = jax.nn.silu(q_g) * up


def _attention(q, k, v, cos, sin, wkc, wvc, wqa, wqg, wov):
    return pl.pallas_call(
        _attn_body,
        grid=(H,),
        in_specs=[
            pl.BlockSpec((S, HD), lambda h: (0, h)),
            pl.BlockSpec((S, HD), lambda h: (0, h)),
            pl.BlockSpec((S, HD), lambda h: (0, h)),
            pl.BlockSpec((S, ROPE), lambda h: (0, 0)),
            pl.BlockSpec((S, ROPE), lambda h: (0, 0)),
            pl.BlockSpec((HD, KVL), lambda h: (0, 0)),
            pl.BlockSpec((HD, KVL), lambda h: (0, 0)),
            pl.BlockSpec((HD, KVL), lambda h: (0, 0)),
            pl.BlockSpec((HD, HD), lambda h: (0, 0)),
            pl.BlockSpec((KVL, HD), lambda h: (0, 0)),
        ],
        out_specs=pl.BlockSpec((S, HD), lambda h: (0, h)),
        out_shape=jax.ShapeDtypeStruct((S, D), jnp.float32),
    )(q, k, v, cos, sin, wkc, wvc, wqa, wqg, wov)


def _shared_body(x_ref, w1_ref, w3_ref, w2_ref, res_ref, o_ref, acc_ref):
    j = pl.program_id(1)

    @pl.when(j == 0)
    def _():
        acc_ref[...] = jnp.zeros_like(acc_ref)

    x = x_ref[...]
    h1 = jnp.dot(x, w1_ref[...], preferred_element_type=jnp.float32)
    h3 = jnp.dot(x, w3_ref[...], preferred_element_type=jnp.float32)
    a = jax.nn.silu(h1) * h3
    acc_ref[...] += jnp.dot(a, w2_ref[...], preferred_element_type=jnp.float32)

    @pl.when(j == NJ_S - 1)
    def _():
        o_ref[...] = acc_ref[...] + res_ref[...]


def _shared_expert_base(x2, sw1, sw3, sw2, hidden):
    bm = 256
    return pl.pallas_call(
        _shared_body,
        grid=(S // bm, NJ_S),
        in_specs=[
            pl.BlockSpec((bm, D), lambda i, j: (i, 0)),
            pl.BlockSpec((D, SH_BN), lambda i, j: (0, j)),
            pl.BlockSpec((D, SH_BN), lambda i, j: (0, j)),
            pl.BlockSpec((SH_BN, D), lambda i, j: (j, 0)),
            pl.BlockSpec((bm, D), lambda i, j: (i, 0)),
        ],
        out_specs=pl.BlockSpec((bm, D), lambda i, j: (i, 0)),
        out_shape=jax.ShapeDtypeStruct((S, D), jnp.float32),
        scratch_shapes=[pltpu.VMEM((bm, D), jnp.float32)],
        compiler_params=pltpu.CompilerParams(
            dimension_semantics=("arbitrary", "arbitrary")),
    )(x2, sw1, sw3, sw2, hidden)


def _gate_body(x_ref, gw_ref, o_ref):
    logits = jnp.dot(x_ref[...], gw_ref[...],
                     preferred_element_type=jnp.float32)
    iota = lax.broadcasted_iota(jnp.int32, logits.shape, 1)
    logits = jnp.where(iota < E, logits, -1e30)
    m = jnp.max(logits, axis=-1, keepdims=True)
    e = jnp.exp(logits - m)
    p = e / jnp.sum(e, axis=-1, keepdims=True)
    v0 = jnp.max(p, axis=-1, keepdims=True)
    i0 = jnp.min(jnp.where(p == v0, iota, 10 ** 9), axis=-1, keepdims=True)
    p2 = jnp.where(iota == i0, -1.0, p)
    v1 = jnp.max(p2, axis=-1, keepdims=True)
    i1 = jnp.min(jnp.where(p2 == v1, iota, 10 ** 9), axis=-1, keepdims=True)
    tot = v0 + v1
    cols = jnp.concatenate(
        [v0 / tot, v1 / tot, i0.astype(jnp.float32), i1.astype(jnp.float32),
         jnp.zeros((x_ref.shape[0], 124), jnp.float32)], axis=-1)
    o_ref[...] = cols


def _router(x2, gate_w):
    gw = jnp.pad(gate_w, ((0, 0), (0, 128 - E)))
    bm = 256
    out = pl.pallas_call(
        _gate_body,
        grid=(S // bm,),
        in_specs=[
            pl.BlockSpec((bm, D), lambda i: (i, 0)),
            pl.BlockSpec((D, 128), lambda i: (0, 0)),
        ],
        out_specs=pl.BlockSpec((bm, 128), lambda i: (i, 0)),
        out_shape=jax.ShapeDtypeStruct((S, 128), jnp.float32),
    )(x2, gw)
    v0, v1 = out[:, 0], out[:, 1]
    i0 = out[:, 2].astype(jnp.int32)
    i1 = out[:, 3].astype(jnp.int32)
    return v0, v1, i0, i1


def _group_body(be_ref, xg_ref, w1_ref, w3_ref, w2_ref, cw_ref, o_ref,
                acc_ref):
    j = pl.program_id(1)

    @pl.when(j == 0)
    def _():
        acc_ref[...] = jnp.zeros_like(acc_ref)

    x = xg_ref[...]
    h1 = jnp.dot(x, w1_ref[0], preferred_element_type=jnp.float32)
    h3 = jnp.dot(x, w3_ref[0], preferred_element_type=jnp.float32)
    a = jax.nn.silu(h1) * h3
    acc_ref[...] += jnp.dot(a, w2_ref[0], preferred_element_type=jnp.float32)

    @pl.when(j == NJ_E - 1)
    def _():
        o_ref[...] = acc_ref[...] * cw_ref[...]


def _grouped_ffn(block_expert, xg, ew1, ew3, ew2, cw):
    grid_spec = pltpu.PrefetchScalarGridSpec(
        num_scalar_prefetch=1,
        grid=(NBLK, NJ_E),
        in_specs=[
            pl.BlockSpec((BM, D), lambda i, j, be: (i, 0)),
            pl.BlockSpec((1, D, GE_BN), lambda i, j, be: (be[i], 0, j)),
            pl.BlockSpec((1, D, GE_BN), lambda i, j, be: (be[i], 0, j)),
            pl.BlockSpec((1, GE_BN, D), lambda i, j, be: (be[i], j, 0)),
            pl.BlockSpec((BM, 1), lambda i, j, be: (i, 0)),
        ],
        out_specs=pl.BlockSpec((BM, D), lambda i, j, be: (i, 0)),
        scratch_shapes=[pltpu.VMEM((BM, D), jnp.float32)],
    )
    return pl.pallas_call(
        _group_body,
        grid_spec=grid_spec,
        out_shape=jax.ShapeDtypeStruct((P, D), jnp.float32),
        compiler_params=pltpu.CompilerParams(
            dimension_semantics=("arbitrary", "arbitrary")),
    )(block_expert, xg, ew1, ew3, ew2, cw.reshape(P, 1))


def _sc_gather_rows(table, idx):
    n = idx.shape[0]
    d = table.shape[1]
    nw = 32
    chunk = 32
    per_w = n // nw
    iters = per_w // chunk
    mesh = plsc.VectorSubcoreMesh(core_axis_name="c", subcore_axis_name="s")

    @functools.partial(
        pl.kernel,
        mesh=mesh,
        out_type=jax.ShapeDtypeStruct((n, d), jnp.float32),
        scratch_types=[
            pltpu.VMEM((chunk,), jnp.int32),
            pltpu.VMEM((chunk, d), jnp.float32),
            pltpu.SemaphoreType.DMA,
        ],
    )
    def gk(table_hbm, idx_hbm, out_hbm, idx_v, rows_v, sem):
        wid = lax.axis_index("s") * 2 + lax.axis_index("c")
        base = wid * per_w

        def body(c, _):
            off = base + c * chunk
            pltpu.sync_copy(idx_hbm.at[pl.ds(off, chunk)], idx_v)
            pltpu.async_copy(table_hbm.at[idx_v], rows_v, sem).wait()
            pltpu.sync_copy(rows_v, out_hbm.at[pl.ds(off, chunk)])
            return ()

        lax.fori_loop(0, iters, body, ())

    return gk(table, idx)


def _combine_body(b_ref, g0_ref, g1_ref, o_ref):
    o_ref[...] = b_ref[...] + g0_ref[...] + g1_ref[...]


def _combine(base, g0, g1):
    bm = 256
    return pl.pallas_call(
        _combine_body,
        grid=(S // bm,),
        in_specs=[
            pl.BlockSpec((bm, D), lambda i: (i, 0)),
            pl.BlockSpec((bm, D), lambda i: (i, 0)),
            pl.BlockSpec((bm, D), lambda i: (i, 0)),
        ],
        out_specs=pl.BlockSpec((bm, D), lambda i: (i, 0)),
        out_shape=jax.ShapeDtypeStruct((S, D), jnp.float32),
    )(base, g0, g1)


def kernel(hidden_states, position_ids, ln1_g, ln1_b, ln2_g, ln2_b, wq, wk,
           wv, wkc, wvc, wqa, wqg, wov, wo, gate_w, ew1, ew2, ew3, sw1, sw2,
           sw3):
    h2d = hidden_states.reshape(S, D)

    xln = _layernorm(h2d, ln1_g, ln1_b)
    q = _matmul(xln, wq)
    k = _matmul(xln, wk)
    v = _matmul(xln, wv)

    inv_freq = 1.0 / (10000.0 ** (jnp.arange(0, ROPE, 2, jnp.float32) / ROPE))
    t = jnp.arange(4096, dtype=jnp.float32)
    freqs = jnp.outer(t, inv_freq)
    emb = jnp.concatenate((freqs, freqs), axis=-1)
    pos = position_ids.reshape(S)
    cos = jnp.cos(emb)[pos]
    sin = jnp.sin(emb)[pos]

    gated = _attention(q, k, v, cos, sin, wkc, wvc, wqa, wqg, wov)
    hidden = _matmul_add(gated, wo, h2d)

    x2 = _layernorm(hidden, ln2_g, ln2_b)
    base = _shared_expert_base(x2, sw1, sw3, sw2, hidden)

    v0, v1, i0, i1 = _router(x2, gate_w)

    e_f = jnp.stack([i0, i1], axis=1).reshape(-1)
    w_f = jnp.stack([v0, v1], axis=1).reshape(-1)
    oh = (e_f[:, None] == jnp.arange(E)[None, :]).astype(jnp.int32)
    ranks = jnp.take_along_axis(jnp.cumsum(oh, axis=0) - oh,
                                e_f[:, None], axis=1)[:, 0]
    counts = jnp.sum(oh, axis=0)
    padded = ((counts + BM - 1) // BM) * BM
    astart = jnp.concatenate([jnp.zeros(1, jnp.int32),
                              jnp.cumsum(padded)[:-1].astype(jnp.int32)])
    dest = astart[e_f] + ranks
    row_ids = jnp.zeros(P, jnp.int32).at[dest].set(
        jnp.arange(S * TOPK, dtype=jnp.int32) // TOPK)
    cw = jnp.zeros(P, jnp.float32).at[dest].set(w_f)
    nblocks_e = padded // BM
    block_expert = jnp.repeat(jnp.arange(E, dtype=jnp.int32), nblocks_e,
                              total_repeat_length=NBLK)

    xg = _sc_gather_rows(x2, row_ids)
    outg = _grouped_ffn(block_expert, xg, ew1, ew3, ew2, cw)
    p01 = jnp.concatenate([dest[0::2], dest[1::2]])
    g01 = _sc_gather_rows(outg, p01)
    y = _combine(base, g01[:S], g01[S:])

    return y.reshape(1, S, D)


# speedup vs baseline: 1.2142x; 1.0046x over previous
import functools

import jax
import jax.numpy as jnp
from jax import lax
from jax.experimental import pallas as pl
from jax.experimental.pallas import tpu as pltpu
from jax.experimental.pallas import tpu_sc as plsc

S = 2048
D = 2048
H = 16
HD = 128
ROPE = 64
KVL = 64
FF = 4096
E = 8
TOPK = 2

BM = 256
P = S * TOPK + E * BM
NBLK = P // BM
GE_BN = 512
NJ_E = FF // GE_BN

SH_BN = 512
NJ_S = FF // SH_BN


def _ln_body(x_ref, g_ref, b_ref, o_ref):
    x = x_ref[...]
    m = jnp.mean(x, axis=-1, keepdims=True)
    v = jnp.mean((x - m) ** 2, axis=-1, keepdims=True)
    o_ref[...] = (x - m) * jax.lax.rsqrt(v + 1e-5) * g_ref[...] + b_ref[...]


def _layernorm(x, g, b):
    bm = 256
    return pl.pallas_call(
        _ln_body,
        grid=(S // bm,),
        in_specs=[
            pl.BlockSpec((bm, D), lambda i: (i, 0)),
            pl.BlockSpec((1, D), lambda i: (0, 0)),
            pl.BlockSpec((1, D), lambda i: (0, 0)),
        ],
        out_specs=pl.BlockSpec((bm, D), lambda i: (i, 0)),
        out_shape=jax.ShapeDtypeStruct((S, D), jnp.float32),
    )(x, g.reshape(1, D), b.reshape(1, D))


def _bf(x):
    return x.astype(jnp.bfloat16)


def _mm_body(x_ref, w_ref, o_ref):
    o_ref[...] = jnp.dot(_bf(x_ref[...]), _bf(w_ref[...]),
                         preferred_element_type=jnp.float32)


def _matmul(x, w):
    m, k = x.shape
    _, n = w.shape
    bm, bn = 256, 512
    return pl.pallas_call(
        _mm_body,
        grid=(m // bm, n // bn),
        in_specs=[
            pl.BlockSpec((bm, k), lambda i, j: (i, 0)),
            pl.BlockSpec((k, bn), lambda i, j: (0, j)),
        ],
        out_specs=pl.BlockSpec((bm, bn), lambda i, j: (i, j)),
        out_shape=jax.ShapeDtypeStruct((m, n), jnp.float32),
    )(x, w)


def _mm_add_body(x_ref, w_ref, r_ref, o_ref):
    o_ref[...] = r_ref[...] + jnp.dot(_bf(x_ref[...]), _bf(w_ref[...]),
                                      preferred_element_type=jnp.float32)


def _matmul_add(x, w, res):
    m, k = x.shape
    _, n = w.shape
    bm, bn = 256, 512
    return pl.pallas_call(
        _mm_add_body,
        grid=(m // bm, n // bn),
        in_specs=[
            pl.BlockSpec((bm, k), lambda i, j: (i, 0)),
            pl.BlockSpec((k, bn), lambda i, j: (0, j)),
            pl.BlockSpec((bm, bn), lambda i, j: (i, j)),
        ],
        out_specs=pl.BlockSpec((bm, bn), lambda i, j: (i, j)),
        out_shape=jax.ShapeDtypeStruct((m, n), jnp.float32),
    )(x, w, res)


def _rot_half(x):
    h = x.shape[-1] // 2
    return jnp.concatenate((-x[..., h:], x[..., :h]), axis=-1)


def _attn_body(q_ref, k_ref, v_ref, cos_ref, sin_ref, wkc_ref, wvc_ref,
               wqa_ref, wqg_ref, wov_ref, o_ref):
    cos = cos_ref[...]
    sin = sin_ref[...]
    q = q_ref[...]
    k = k_ref[...]
    qr = q[:, :ROPE] * cos + _rot_half(q[:, :ROPE]) * sin
    kr = k[:, :ROPE] * cos + _rot_half(k[:, :ROPE]) * sin
    qf = jnp.concatenate([qr, q[:, ROPE:]], axis=-1)
    kf = jnp.concatenate([kr, k[:, ROPE:]], axis=-1)
    k_c = jnp.dot(_bf(kf), _bf(wkc_ref[...]),
                  preferred_element_type=jnp.float32)
    v_c = jnp.dot(_bf(v_ref[...]), _bf(wvc_ref[...]),
                  preferred_element_type=jnp.float32)
    q_a = jnp.dot(_bf(qf), _bf(wqa_ref[...]),
                  preferred_element_type=jnp.float32)
    q_g = jnp.dot(_bf(qf), _bf(wqg_ref[...]),
                  preferred_element_type=jnp.float32)
    s = lax.dot_general(_bf(q_a), _bf(k_c), (((1,), (1,)), ((), ())),
                        preferred_element_type=jnp.float32)
    s = s * (1.0 / (KVL ** 0.5))
    s = s - jnp.max(s, axis=-1, keepdims=True)
    p = jnp.exp(s)
    p = p / jnp.sum(p, axis=-1, keepdims=True)
    o_c = jnp.dot(_bf(p), _bf(v_c), preferred_element_type=jnp.float32)
    up = jnp.dot(_bf(o_c), _bf(wov_ref[...]),
                 preferred_element_type=jnp.float32)
    o_ref[...] = jax.nn.silu(q_g) * up


def _attention(q, k, v, cos, sin, wkc, wvc, wqa, wqg, wov):
    return pl.pallas_call(
        _attn_body,
        grid=(H,),
        in_specs=[
            pl.BlockSpec((S, HD), lambda h: (0, h)),
            pl.BlockSpec((S, HD), lambda h: (0, h)),
            pl.BlockSpec((S, HD), lambda h: (0, h)),
            pl.BlockSpec((S, ROPE), lambda h: (0, 0)),
            pl.BlockSpec((S, ROPE), lambda h: (0, 0)),
            pl.BlockSpec((HD, KVL), lambda h: (0, 0)),
            pl.BlockSpec((HD, KVL), lambda h: (0, 0)),
            pl.BlockSpec((HD, KVL), lambda h: (0, 0)),
            pl.BlockSpec((HD, HD), lambda h: (0, 0)),
            pl.BlockSpec((KVL, HD), lambda h: (0, 0)),
        ],
        out_specs=pl.BlockSpec((S, HD), lambda h: (0, h)),
        out_shape=jax.ShapeDtypeStruct((S, D), jnp.float32),
    )(q, k, v, cos, sin, wkc, wvc, wqa, wqg, wov)


def _shared_body(x_ref, w1_ref, w3_ref, w2_ref, res_ref, o_ref, acc_ref):
    j = pl.program_id(1)

    @pl.when(j == 0)
    def _():
        acc_ref[...] = jnp.zeros_like(acc_ref)

    x = _bf(x_ref[...])
    h1 = jnp.dot(x, _bf(w1_ref[...]), preferred_element_type=jnp.float32)
    h3 = jnp.dot(x, _bf(w3_ref[...]), preferred_element_type=jnp.float32)
    a = _bf(jax.nn.silu(h1) * h3)
    acc_ref[...] += jnp.dot(a, _bf(w2_ref[...]),
                            preferred_element_type=jnp.float32)

    @pl.when(j == NJ_S - 1)
    def _():
        o_ref[...] = acc_ref[...] + res_ref[...]


def _shared_expert_base(x2, sw1, sw3, sw2, hidden):
    bm = 256
    return pl.pallas_call(
        _shared_body,
        grid=(S // bm, NJ_S),
        in_specs=[
            pl.BlockSpec((bm, D), lambda i, j: (i, 0)),
            pl.BlockSpec((D, SH_BN), lambda i, j: (0, j)),
            pl.BlockSpec((D, SH_BN), lambda i, j: (0, j)),
            pl.BlockSpec((SH_BN, D), lambda i, j: (j, 0)),
            pl.BlockSpec((bm, D), lambda i, j: (i, 0)),
        ],
        out_specs=pl.BlockSpec((bm, D), lambda i, j: (i, 0)),
        out_shape=jax.ShapeDtypeStruct((S, D), jnp.float32),
        scratch_shapes=[pltpu.VMEM((bm, D), jnp.float32)],
        compiler_params=pltpu.CompilerParams(
            dimension_semantics=("arbitrary", "arbitrary")),
    )(x2, sw1, sw3, sw2, hidden)


def _gate_body(x_ref, gw_ref, o_ref):
    logits = jnp.dot(x_ref[...], gw_ref[...],
                     preferred_element_type=jnp.float32)
    iota = lax.broadcasted_iota(jnp.int32, logits.shape, 1)
    logits = jnp.where(iota < E, logits, -1e30)
    m = jnp.max(logits, axis=-1, keepdims=True)
    e = jnp.exp(logits - m)
    p = e / jnp.sum(e, axis=-1, keepdims=True)
    v0 = jnp.max(p, axis=-1, keepdims=True)
    i0 = jnp.min(jnp.where(p == v0, iota, 10 ** 9), axis=-1, keepdims=True)
    p2 = jnp.where(iota == i0, -1.0, p)
    v1 = jnp.max(p2, axis=-1, keepdims=True)
    i1 = jnp.min(jnp.where(p2 == v1, iota, 10 ** 9), axis=-1, keepdims=True)
    tot = v0 + v1
    cols = jnp.concatenate(
        [v0 / tot, v1 / tot, i0.astype(jnp.float32), i1.astype(jnp.float32),
         jnp.zeros((x_ref.shape[0], 124), jnp.float32)], axis=-1)
    o_ref[...] = cols


def _router(x2, gate_w):
    gw = jnp.pad(gate_w, ((0, 0), (0, 128 - E)))
    bm = 256
    out = pl.pallas_call(
        _gate_body,
        grid=(S // bm,),
        in_specs=[
            pl.BlockSpec((bm, D), lambda i: (i, 0)),
            pl.BlockSpec((D, 128), lambda i: (0, 0)),
        ],
        out_specs=pl.BlockSpec((bm, 128), lambda i: (i, 0)),
        out_shape=jax.ShapeDtypeStruct((S, 128), jnp.float32),
    )(x2, gw)
    v0, v1 = out[:, 0], out[:, 1]
    i0 = out[:, 2].astype(jnp.int32)
    i1 = out[:, 3].astype(jnp.int32)
    return v0, v1, i0, i1


def _group_body(be_ref, xg_ref, w1_ref, w3_ref, w2_ref, cw_ref, o_ref,
                acc_ref):
    j = pl.program_id(1)

    @pl.when(j == 0)
    def _():
        acc_ref[...] = jnp.zeros_like(acc_ref)

    x = _bf(xg_ref[...])
    h1 = jnp.dot(x, _bf(w1_ref[0]), preferred_element_type=jnp.float32)
    h3 = jnp.dot(x, _bf(w3_ref[0]), preferred_element_type=jnp.float32)
    a = _bf(jax.nn.silu(h1) * h3)
    acc_ref[...] += jnp.dot(a, _bf(w2_ref[0]),
                            preferred_element_type=jnp.float32)

    @pl.when(j == NJ_E - 1)
    def _():
        o_ref[...] = acc_ref[...] * cw_ref[...]


def _grouped_ffn(block_expert, xg, ew1, ew3, ew2, cw):
    grid_spec = pltpu.PrefetchScalarGridSpec(
        num_scalar_prefetch=1,
        grid=(NBLK, NJ_E),
        in_specs=[
            pl.BlockSpec((BM, D), lambda i, j, be: (i, 0)),
            pl.BlockSpec((1, D, GE_BN), lambda i, j, be: (be[i], 0, j)),
            pl.BlockSpec((1, D, GE_BN), lambda i, j, be: (be[i], 0, j)),
            pl.BlockSpec((1, GE_BN, D), lambda i, j, be: (be[i], j, 0)),
            pl.BlockSpec((BM, 1), lambda i, j, be: (i, 0)),
        ],
        out_specs=pl.BlockSpec((BM, D), lambda i, j, be: (i, 0)),
        scratch_shapes=[pltpu.VMEM((BM, D), jnp.float32)],
    )
    return pl.pallas_call(
        _group_body,
        grid_spec=grid_spec,
        out_shape=jax.ShapeDtypeStruct((P, D), jnp.float32),
        compiler_params=pltpu.CompilerParams(
            dimension_semantics=("arbitrary", "arbitrary")),
    )(block_expert, xg, ew1, ew3, ew2, cw.reshape(P, 1))


def _sc_gather_rows(table, idx):
    n = idx.shape[0]
    d = table.shape[1]
    nw = 32
    chunk = 32
    per_w = n // nw
    iters = per_w // chunk
    mesh = plsc.VectorSubcoreMesh(core_axis_name="c", subcore_axis_name="s")

    @functools.partial(
        pl.kernel,
        mesh=mesh,
        out_type=jax.ShapeDtypeStruct((n, d), jnp.float32),
        scratch_types=[
            pltpu.VMEM((chunk,), jnp.int32),
            pltpu.VMEM((chunk, d), jnp.float32),
            pltpu.SemaphoreType.DMA,
        ],
    )
    def gk(table_hbm, idx_hbm, out_hbm, idx_v, rows_v, sem):
        wid = lax.axis_index("s") * 2 + lax.axis_index("c")
        base = wid * per_w

        def body(c, _):
            off = base + c * chunk
            pltpu.sync_copy(idx_hbm.at[pl.ds(off, chunk)], idx_v)
            pltpu.async_copy(table_hbm.at[idx_v], rows_v, sem).wait()
            pltpu.sync_copy(rows_v, out_hbm.at[pl.ds(off, chunk)])
            return ()

        lax.fori_loop(0, iters, body, ())

    return gk(table, idx)


def _combine_body(b_ref, g0_ref, g1_ref, o_ref):
    o_ref[...] = b_ref[...] + g0_ref[...] + g1_ref[...]


def _combine(base, g0, g1):
    bm = 256
    return pl.pallas_call(
        _combine_body,
        grid=(S // bm,),
        in_specs=[
            pl.BlockSpec((bm, D), lambda i: (i, 0)),
            pl.BlockSpec((bm, D), lambda i: (i, 0)),
            pl.BlockSpec((bm, D), lambda i: (i, 0)),
        ],
        out_specs=pl.BlockSpec((bm, D), lambda i: (i, 0)),
        out_shape=jax.ShapeDtypeStruct((S, D), jnp.float32),
    )(base, g0, g1)


def kernel(hidden_states, position_ids, ln1_g, ln1_b, ln2_g, ln2_b, wq, wk,
           wv, wkc, wvc, wqa, wqg, wov, wo, gate_w, ew1, ew2, ew3, sw1, sw2,
           sw3):
    h2d = hidden_states.reshape(S, D)

    xln = _layernorm(h2d, ln1_g, ln1_b)
    q = _matmul(xln, wq)
    k = _matmul(xln, wk)
    v = _matmul(xln, wv)

    inv_freq = 1.0 / (10000.0 ** (jnp.arange(0, ROPE, 2, jnp.float32) / ROPE))
    t = jnp.arange(4096, dtype=jnp.float32)
    freqs = jnp.outer(t, inv_freq)
    emb = jnp.concatenate((freqs, freqs), axis=-1)
    pos = position_ids.reshape(S)
    cos = jnp.cos(emb)[pos]
    sin = jnp.sin(emb)[pos]

    gated = _attention(q, k, v, cos, sin, wkc, wvc, wqa, wqg, wov)
    hidden = _matmul_add(gated, wo, h2d)

    x2 = _layernorm(hidden, ln2_g, ln2_b)
    base = _shared_expert_base(x2, sw1, sw3, sw2, hidden)

    v0, v1, i0, i1 = _router(x2, gate_w)

    e_f = jnp.stack([i0, i1], axis=1).reshape(-1)
    w_f = jnp.stack([v0, v1], axis=1).reshape(-1)
    oh = (e_f[:, None] == jnp.arange(E)[None, :]).astype(jnp.int32)
    ranks = jnp.take_along_axis(jnp.cumsum(oh, axis=0) - oh,
                                e_f[:, None], axis=1)[:, 0]
    counts = jnp.sum(oh, axis=0)
    padded = ((counts + BM - 1) // BM) * BM
    astart = jnp.concatenate([jnp.zeros(1, jnp.int32),
                              jnp.cumsum(padded)[:-1].astype(jnp.int32)])
    dest = astart[e_f] + ranks
    row_ids = jnp.zeros(P, jnp.int32).at[dest].set(
        jnp.arange(S * TOPK, dtype=jnp.int32) // TOPK)
    cw = jnp.zeros(P, jnp.float32).at[dest].set(w_f)
    nblocks_e = padded // BM
    block_expert = jnp.repeat(jnp.arange(E, dtype=jnp.int32), nblocks_e,
                              total_repeat_length=NBLK)

    xg = _sc_gather_rows(x2, row_ids)
    outg = _grouped_ffn(block_expert, xg, ew1, ew3, ew2, cw)
    p01 = jnp.concatenate([dest[0::2], dest[1::2]])
    g01 = _sc_gather_rows(outg, p01)
    y = _combine(base, g01[:S], g01[S:])

    return y.reshape(1, S, D)


# speedup vs baseline: 1.4522x; 1.1961x over previous
import functools

import jax
import jax.numpy as jnp
from jax import lax
from jax.experimental import pallas as pl
from jax.experimental.pallas import tpu as pltpu
from jax.experimental.pallas import tpu_sc as plsc

S = 2048
D = 2048
H = 16
HD = 128
ROPE = 64
KVL = 64
FF = 4096
E = 8
TOPK = 2

BM = 512
P = S * TOPK + E * BM
NBLK = P // BM
GE_BN = 512
NJ_E = FF // GE_BN

SH_BN = 512
NJ_S = FF // SH_BN


def _ln_body(x_ref, g_ref, b_ref, o_ref):
    x = x_ref[...]
    m = jnp.mean(x, axis=-1, keepdims=True)
    v = jnp.mean((x - m) ** 2, axis=-1, keepdims=True)
    o_ref[...] = (x - m) * jax.lax.rsqrt(v + 1e-5) * g_ref[...] + b_ref[...]


def _layernorm(x, g, b):
    bm = 256
    return pl.pallas_call(
        _ln_body,
        grid=(S // bm,),
        in_specs=[
            pl.BlockSpec((bm, D), lambda i: (i, 0)),
            pl.BlockSpec((1, D), lambda i: (0, 0)),
            pl.BlockSpec((1, D), lambda i: (0, 0)),
        ],
        out_specs=pl.BlockSpec((bm, D), lambda i: (i, 0)),
        out_shape=jax.ShapeDtypeStruct((S, D), jnp.float32),
    )(x, g.reshape(1, D), b.reshape(1, D))


def _bf(x):
    return x.astype(jnp.bfloat16)


def _mm_body(x_ref, w_ref, o_ref):
    o_ref[...] = jnp.dot(_bf(x_ref[...]), _bf(w_ref[...]),
                         preferred_element_type=jnp.float32)


def _matmul(x, w):
    m, k = x.shape
    _, n = w.shape
    bm, bn = 256, 1024
    return pl.pallas_call(
        _mm_body,
        grid=(m // bm, n // bn),
        in_specs=[
            pl.BlockSpec((bm, k), lambda i, j: (i, 0)),
            pl.BlockSpec((k, bn), lambda i, j: (0, j)),
        ],
        out_specs=pl.BlockSpec((bm, bn), lambda i, j: (i, j)),
        out_shape=jax.ShapeDtypeStruct((m, n), jnp.float32),
    )(x, w)


def _mm_add_body(x_ref, w_ref, r_ref, o_ref):
    o_ref[...] = r_ref[...] + jnp.dot(_bf(x_ref[...]), _bf(w_ref[...]),
                                      preferred_element_type=jnp.float32)


def _matmul_add(x, w, res):
    m, k = x.shape
    _, n = w.shape
    bm, bn = 256, 1024
    return pl.pallas_call(
        _mm_add_body,
        grid=(m // bm, n // bn),
        in_specs=[
            pl.BlockSpec((bm, k), lambda i, j: (i, 0)),
            pl.BlockSpec((k, bn), lambda i, j: (0, j)),
            pl.BlockSpec((bm, bn), lambda i, j: (i, j)),
        ],
        out_specs=pl.BlockSpec((bm, bn), lambda i, j: (i, j)),
        out_shape=jax.ShapeDtypeStruct((m, n), jnp.float32),
    )(x, w, res)


def _rot_half(x):
    h = x.shape[-1] // 2
    return jnp.concatenate((-x[..., h:], x[..., :h]), axis=-1)


def _attn_body(q_ref, k_ref, v_ref, cos_ref, sin_ref, wkc_ref, wvc_ref,
               wqa_ref, wqg_ref, wov_ref, o_ref):
    cos = cos_ref[...]
    sin = sin_ref[...]
    q = q_ref[...]
    k = k_ref[...]
    qr = q[:, :ROPE] * cos + _rot_half(q[:, :ROPE]) * sin
    kr = k[:, :ROPE] * cos + _rot_half(k[:, :ROPE]) * sin
    qf = jnp.concatenate([qr, q[:, ROPE:]], axis=-1)
    kf = jnp.concatenate([kr, k[:, ROPE:]], axis=-1)
    k_c = jnp.dot(_bf(kf), _bf(wkc_ref[...]),
                  preferred_element_type=jnp.float32)
    v_c = jnp.dot(_bf(v_ref[...]), _bf(wvc_ref[...]),
                  preferred_element_type=jnp.float32)
    q_a = jnp.dot(_bf(qf), _bf(wqa_ref[...]),
                  preferred_element_type=jnp.float32)
    q_g = jnp.dot(_bf(qf), _bf(wqg_ref[...]),
                  preferred_element_type=jnp.float32)
    s = lax.dot_general(_bf(q_a), _bf(k_c), (((1,), (1,)), ((), ())),
                        preferred_element_type=jnp.float32)
    s = s * (1.0 / (KVL ** 0.5))
    s = s - jnp.max(s, axis=-1, keepdims=True)
    p = jnp.exp(s)
    p = p / jnp.sum(p, axis=-1, keepdims=True)
    o_c = jnp.dot(_bf(p), _bf(v_c), preferred_element_type=jnp.float32)
    up = jnp.dot(_bf(o_c), _bf(wov_ref[...]),
                 preferred_element_type=jnp.float32)
    o_ref[...] = jax.nn.silu(q_g) * up


def _attention(q, k, v, cos, sin, wkc, wvc, wqa, wqg, wov):
    return pl.pallas_call(
        _attn_body,
        grid=(H,),
        in_specs=[
            pl.BlockSpec((S, HD), lambda h: (0, h)),
            pl.BlockSpec((S, HD), lambda h: (0, h)),
            pl.BlockSpec((S, HD), lambda h: (0, h)),
            pl.BlockSpec((S, ROPE), lambda h: (0, 0)),
            pl.BlockSpec((S, ROPE), lambda h: (0, 0)),
            pl.BlockSpec((HD, KVL), lambda h: (0, 0)),
            pl.BlockSpec((HD, KVL), lambda h: (0, 0)),
            pl.BlockSpec((HD, KVL), lambda h: (0, 0)),
            pl.BlockSpec((HD, HD), lambda h: (0, 0)),
            pl.BlockSpec((KVL, HD), lambda h: (0, 0)),
        ],
        out_specs=pl.BlockSpec((S, HD), lambda h: (0, h)),
        out_shape=jax.ShapeDtypeStruct((S, D), jnp.float32),
    )(q, k, v, cos, sin, wkc, wvc, wqa, wqg, wov)


def _shared_body(x_ref, w1_ref, w3_ref, w2_ref, res_ref, o_ref, acc_ref):
    j = pl.program_id(1)

    @pl.when(j == 0)
    def _():
        acc_ref[...] = jnp.zeros_like(acc_ref)

    x = _bf(x_ref[...])
    h1 = jnp.dot(x, _bf(w1_ref[...]), preferred_element_type=jnp.float32)
    h3 = jnp.dot(x, _bf(w3_ref[...]), preferred_element_type=jnp.float32)
    a = _bf(jax.nn.silu(h1) * h3)
    acc_ref[...] += jnp.dot(a, _bf(w2_ref[...]),
                            preferred_element_type=jnp.float32)

    @pl.when(j == NJ_S - 1)
    def _():
        o_ref[...] = acc_ref[...] + res_ref[...]


def _shared_expert_base(x2, sw1, sw3, sw2, hidden):
    bm = 256
    return pl.pallas_call(
        _shared_body,
        grid=(S // bm, NJ_S),
        in_specs=[
            pl.BlockSpec((bm, D), lambda i, j: (i, 0)),
            pl.BlockSpec((D, SH_BN), lambda i, j: (0, j)),
            pl.BlockSpec((D, SH_BN), lambda i, j: (0, j)),
            pl.BlockSpec((SH_BN, D), lambda i, j: (j, 0)),
            pl.BlockSpec((bm, D), lambda i, j: (i, 0)),
        ],
        out_specs=pl.BlockSpec((bm, D), lambda i, j: (i, 0)),
        out_shape=jax.ShapeDtypeStruct((S, D), jnp.float32),
        scratch_shapes=[pltpu.VMEM((bm, D), jnp.float32)],
        compiler_params=pltpu.CompilerParams(
            dimension_semantics=("arbitrary", "arbitrary")),
    )(x2, sw1, sw3, sw2, hidden)


def _gate_body(x_ref, gw_ref, o_ref):
    logits = jnp.dot(x_ref[...], gw_ref[...],
                     preferred_element_type=jnp.float32)
    iota = lax.broadcasted_iota(jnp.int32, logits.shape, 1)
    logits = jnp.where(iota < E, logits, -1e30)
    m = jnp.max(logits, axis=-1, keepdims=True)
    e = jnp.exp(logits - m)
    p = e / jnp.sum(e, axis=-1, keepdims=True)
    v0 = jnp.max(p, axis=-1, keepdims=True)
    i0 = jnp.min(jnp.where(p == v0, iota, 10 ** 9), axis=-1, keepdims=True)
    p2 = jnp.where(iota == i0, -1.0, p)
    v1 = jnp.max(p2, axis=-1, keepdims=True)
    i1 = jnp.min(jnp.where(p2 == v1, iota, 10 ** 9), axis=-1, keepdims=True)
    tot = v0 + v1
    cols = jnp.concatenate(
        [v0 / tot, v1 / tot, i0.astype(jnp.float32), i1.astype(jnp.float32),
         jnp.zeros((x_ref.shape[0], 124), jnp.float32)], axis=-1)
    o_ref[...] = cols


def _router(x2, gate_w):
    gw = jnp.pad(gate_w, ((0, 0), (0, 128 - E)))
    bm = 256
    out = pl.pallas_call(
        _gate_body,
        grid=(S // bm,),
        in_specs=[
            pl.BlockSpec((bm, D), lambda i: (i, 0)),
            pl.BlockSpec((D, 128), lambda i: (0, 0)),
        ],
        out_specs=pl.BlockSpec((bm, 128), lambda i: (i, 0)),
        out_shape=jax.ShapeDtypeStruct((S, 128), jnp.float32),
    )(x2, gw)
    v0, v1 = out[:, 0], out[:, 1]
    i0 = out[:, 2].astype(jnp.int32)
    i1 = out[:, 3].astype(jnp.int32)
    return v0, v1, i0, i1


def _group_body(be_ref, xg_ref, w1_ref, w3_ref, w2_ref, cw_ref, o_ref,
                acc_ref):
    j = pl.program_id(1)

    @pl.when(j == 0)
    def _():
        acc_ref[...] = jnp.zeros_like(acc_ref)

    x = _bf(xg_ref[...])
    h1 = jnp.dot(x, _bf(w1_ref[0]), preferred_element_type=jnp.float32)
    h3 = jnp.dot(x, _bf(w3_ref[0]), preferred_element_type=jnp.float32)
    a = _bf(jax.nn.silu(h1) * h3)
    acc_ref[...] += jnp.dot(a, _bf(w2_ref[0]),
                            preferred_element_type=jnp.float32)

    @pl.when(j == NJ_E - 1)
    def _():
        o_ref[...] = acc_ref[...] * cw_ref[...]


def _grouped_ffn(block_expert, xg, ew1, ew3, ew2, cw):
    grid_spec = pltpu.PrefetchScalarGridSpec(
        num_scalar_prefetch=1,
        grid=(NBLK, NJ_E),
        in_specs=[
            pl.BlockSpec((BM, D), lambda i, j, be: (i, 0)),
            pl.BlockSpec((1, D, GE_BN), lambda i, j, be: (be[i], 0, j)),
            pl.BlockSpec((1, D, GE_BN), lambda i, j, be: (be[i], 0, j)),
            pl.BlockSpec((1, GE_BN, D), lambda i, j, be: (be[i], j, 0)),
            pl.BlockSpec((BM, 1), lambda i, j, be: (i, 0)),
        ],
        out_specs=pl.BlockSpec((BM, D), lambda i, j, be: (i, 0)),
        scratch_shapes=[pltpu.VMEM((BM, D), jnp.float32)],
    )
    return pl.pallas_call(
        _group_body,
        grid_spec=grid_spec,
        out_shape=jax.ShapeDtypeStruct((P, D), jnp.float32),
        compiler_params=pltpu.CompilerParams(
            dimension_semantics=("arbitrary", "arbitrary")),
    )(block_expert, xg, ew1, ew3, ew2, cw.reshape(P, 1))


def _sc_gather_rows(table, idx):
    n = idx.shape[0]
    d = table.shape[1]
    nw = 32
    chunk = 32
    per_w = n // nw
    iters = per_w // chunk
    mesh = plsc.VectorSubcoreMesh(core_axis_name="c", subcore_axis_name="s")

    @functools.partial(
        pl.kernel,
        mesh=mesh,
        out_type=jax.ShapeDtypeStruct((n, d), jnp.float32),
        scratch_types=[
            pltpu.VMEM((chunk,), jnp.int32),
            pltpu.VMEM((chunk, d), jnp.float32),
            pltpu.SemaphoreType.DMA,
        ],
    )
    def gk(table_hbm, idx_hbm, out_hbm, idx_v, rows_v, sem):
        wid = lax.axis_index("s") * 2 + lax.axis_index("c")
        base = wid * per_w

        def body(c, _):
            off = base + c * chunk
            pltpu.sync_copy(idx_hbm.at[pl.ds(off, chunk)], idx_v)
            pltpu.async_copy(table_hbm.at[idx_v], rows_v, sem).wait()
            pltpu.sync_copy(rows_v, out_hbm.at[pl.ds(off, chunk)])
            return ()

        lax.fori_loop(0, iters, body, ())

    return gk(table, idx)


def _combine_body(b_ref, g0_ref, g1_ref, o_ref):
    o_ref[...] = b_ref[...] + g0_ref[...] + g1_ref[...]


def _combine(base, g0, g1):
    bm = 256
    return pl.pallas_call(
        _combine_body,
        grid=(S // bm,),
        in_specs=[
            pl.BlockSpec((bm, D), lambda i: (i, 0)),
            pl.BlockSpec((bm, D), lambda i: (i, 0)),
            pl.BlockSpec((bm, D), lambda i: (i, 0)),
        ],
        out_specs=pl.BlockSpec((bm, D), lambda i: (i, 0)),
        out_shape=jax.ShapeDtypeStruct((S, D), jnp.float32),
    )(base, g0, g1)


def kernel(hidden_states, position_ids, ln1_g, ln1_b, ln2_g, ln2_b, wq, wk,
           wv, wkc, wvc, wqa, wqg, wov, wo, gate_w, ew1, ew2, ew3, sw1, sw2,
           sw3):
    h2d = hidden_states.reshape(S, D)

    xln = _layernorm(h2d, ln1_g, ln1_b)
    q = _matmul(xln, wq)
    k = _matmul(xln, wk)
    v = _matmul(xln, wv)

    inv_freq = 1.0 / (10000.0 ** (jnp.arange(0, ROPE, 2, jnp.float32) / ROPE))
    t = jnp.arange(4096, dtype=jnp.float32)
    freqs = jnp.outer(t, inv_freq)
    emb = jnp.concatenate((freqs, freqs), axis=-1)
    pos = position_ids.reshape(S)
    cos = jnp.cos(emb)[pos]
    sin = jnp.sin(emb)[pos]

    gated = _attention(q, k, v, cos, sin, wkc, wvc, wqa, wqg, wov)
    hidden = _matmul_add(gated, wo, h2d)

    x2 = _layernorm(hidden, ln2_g, ln2_b)
    base = _shared_expert_base(x2, sw1, sw3, sw2, hidden)

    v0, v1, i0, i1 = _router(x2, gate_w)

    e_f = jnp.stack([i0, i1], axis=1).reshape(-1)
    w_f = jnp.stack([v0, v1], axis=1).reshape(-1)
    oh = (e_f[:, None] == jnp.arange(E)[None, :]).astype(jnp.int32)
    ranks = jnp.take_along_axis(jnp.cumsum(oh, axis=0) - oh,
                                e_f[:, None], axis=1)[:, 0]
    counts = jnp.sum(oh, axis=0)
    padded = ((counts + BM - 1) // BM) * BM
    astart = jnp.concatenate([jnp.zeros(1, jnp.int32),
                              jnp.cumsum(padded)[:-1].astype(jnp.int32)])
    dest = astart[e_f] + ranks
    row_ids = (jnp.arange(P, dtype=jnp.int32) % S).at[dest].set(
        jnp.arange(S * TOPK, dtype=jnp.int32) // TOPK)
    cw = jnp.zeros(P, jnp.float32).at[dest].set(w_f)
    nblocks_e = padded // BM
    block_expert = jnp.repeat(jnp.arange(E, dtype=jnp.int32), nblocks_e,
                              total_repeat_length=NBLK)

    xg = _sc_gather_rows(x2, row_ids)
    outg = _grouped_ffn(block_expert, xg, ew1, ew3, ew2, cw)
    p01 = jnp.concatenate([dest[0::2], dest[1::2]])
    g01 = _sc_gather_rows(outg, p01)
    y = _combine(base, g01[:S], g01[S:])

    return y.reshape(1, S, D)


# speedup vs baseline: 1.6789x; 1.1561x over previous
import functools

import jax
import jax.numpy as jnp
from jax import lax
from jax.experimental import pallas as pl
from jax.experimental.pallas import tpu as pltpu
from jax.experimental.pallas import tpu_sc as plsc

S = 2048
D = 2048
H = 16
HD = 128
ROPE = 64
KVL = 64
FF = 4096
E = 8
TOPK = 2

BM = 512
P = S * TOPK + E * BM
NBLK = P // BM
GE_BN = 512
NJ_E = FF // GE_BN

SH_BN = 512
NJ_S = FF // SH_BN


def _ln_body(x_ref, g_ref, b_ref, o_ref):
    x = x_ref[...]
    m = jnp.mean(x, axis=-1, keepdims=True)
    v = jnp.mean((x - m) ** 2, axis=-1, keepdims=True)
    o_ref[...] = (x - m) * jax.lax.rsqrt(v + 1e-5) * g_ref[...] + b_ref[...]


def _layernorm(x, g, b):
    bm = 256
    return pl.pallas_call(
        _ln_body,
        grid=(S // bm,),
        in_specs=[
            pl.BlockSpec((bm, D), lambda i: (i, 0)),
            pl.BlockSpec((1, D), lambda i: (0, 0)),
            pl.BlockSpec((1, D), lambda i: (0, 0)),
        ],
        out_specs=pl.BlockSpec((bm, D), lambda i: (i, 0)),
        out_shape=jax.ShapeDtypeStruct((S, D), jnp.float32),
    )(x, g.reshape(1, D), b.reshape(1, D))


def _bf(x):
    return x.astype(jnp.bfloat16)


def _mm_body(x_ref, w_ref, o_ref):
    o_ref[...] = jnp.dot(_bf(x_ref[...]), _bf(w_ref[...]),
                         preferred_element_type=jnp.float32)


def _matmul(x, w):
    m, k = x.shape
    _, n = w.shape
    bm, bn = 256, 1024
    return pl.pallas_call(
        _mm_body,
        grid=(m // bm, n // bn),
        in_specs=[
            pl.BlockSpec((bm, k), lambda i, j: (i, 0)),
            pl.BlockSpec((k, bn), lambda i, j: (0, j)),
        ],
        out_specs=pl.BlockSpec((bm, bn), lambda i, j: (i, j)),
        out_shape=jax.ShapeDtypeStruct((m, n), jnp.float32),
    )(x, w)


def _mm_add_body(x_ref, w_ref, r_ref, o_ref):
    o_ref[...] = r_ref[...] + jnp.dot(_bf(x_ref[...]), _bf(w_ref[...]),
                                      preferred_element_type=jnp.float32)


def _matmul_add(x, w, res):
    m, k = x.shape
    _, n = w.shape
    bm, bn = 256, 1024
    return pl.pallas_call(
        _mm_add_body,
        grid=(m // bm, n // bn),
        in_specs=[
            pl.BlockSpec((bm, k), lambda i, j: (i, 0)),
            pl.BlockSpec((k, bn), lambda i, j: (0, j)),
            pl.BlockSpec((bm, bn), lambda i, j: (i, j)),
        ],
        out_specs=pl.BlockSpec((bm, bn), lambda i, j: (i, j)),
        out_shape=jax.ShapeDtypeStruct((m, n), jnp.float32),
    )(x, w, res)


def _rot_half(x):
    h = x.shape[-1] // 2
    return jnp.concatenate((-x[..., h:], x[..., :h]), axis=-1)


def _attn_body(q_ref, k_ref, v_ref, cos_ref, sin_ref, wkc_ref, wvc_ref,
               wqa_ref, wqg_ref, wov_ref, o_ref):
    cos = cos_ref[...]
    sin = sin_ref[...]
    q = q_ref[...]
    k = k_ref[...]
    qr = q[:, :ROPE] * cos + _rot_half(q[:, :ROPE]) * sin
    kr = k[:, :ROPE] * cos + _rot_half(k[:, :ROPE]) * sin
    qf = jnp.concatenate([qr, q[:, ROPE:]], axis=-1)
    kf = jnp.concatenate([kr, k[:, ROPE:]], axis=-1)
    k_c = jnp.dot(_bf(kf), _bf(wkc_ref[...]),
                  preferred_element_type=jnp.float32)
    v_c = jnp.dot(_bf(v_ref[...]), _bf(wvc_ref[...]),
                  preferred_element_type=jnp.float32)
    q_a = jnp.dot(_bf(qf), _bf(wqa_ref[...]),
                  preferred_element_type=jnp.float32)
    q_g = jnp.dot(_bf(qf), _bf(wqg_ref[...]),
                  preferred_element_type=jnp.float32)
    s = lax.dot_general(_bf(q_a), _bf(k_c), (((1,), (1,)), ((), ())),
                        preferred_element_type=jnp.float32)
    s = s * (1.0 / (KVL ** 0.5))
    s = s - jnp.max(s, axis=-1, keepdims=True)
    p = jnp.exp(s)
    p = p / jnp.sum(p, axis=-1, keepdims=True)
    o_c = jnp.dot(_bf(p), _bf(v_c), preferred_element_type=jnp.float32)
    up = jnp.dot(_bf(o_c), _bf(wov_ref[...]),
                 preferred_element_type=jnp.float32)
    o_ref[...] = jax.nn.silu(q_g) * up


def _attention(q, k, v, cos, sin, wkc, wvc, wqa, wqg, wov):
    return pl.pallas_call(
        _attn_body,
        grid=(H,),
        in_specs=[
            pl.BlockSpec((S, HD), lambda h: (0, h)),
            pl.BlockSpec((S, HD), lambda h: (0, h)),
            pl.BlockSpec((S, HD), lambda h: (0, h)),
            pl.BlockSpec((S, ROPE), lambda h: (0, 0)),
            pl.BlockSpec((S, ROPE), lambda h: (0, 0)),
            pl.BlockSpec((HD, KVL), lambda h: (0, 0)),
            pl.BlockSpec((HD, KVL), lambda h: (0, 0)),
            pl.BlockSpec((HD, KVL), lambda h: (0, 0)),
            pl.BlockSpec((HD, HD), lambda h: (0, 0)),
            pl.BlockSpec((KVL, HD), lambda h: (0, 0)),
        ],
        out_specs=pl.BlockSpec((S, HD), lambda h: (0, h)),
        out_shape=jax.ShapeDtypeStruct((S, D), jnp.float32),
    )(q, k, v, cos, sin, wkc, wvc, wqa, wqg, wov)


def _shared_body(x_ref, w1_ref, w3_ref, w2_ref, res_ref, o_ref, acc_ref):
    j = pl.program_id(1)

    @pl.when(j == 0)
    def _():
        acc_ref[...] = jnp.zeros_like(acc_ref)

    x = _bf(x_ref[...])
    h1 = jnp.dot(x, _bf(w1_ref[...]), preferred_element_type=jnp.float32)
    h3 = jnp.dot(x, _bf(w3_ref[...]), preferred_element_type=jnp.float32)
    a = _bf(jax.nn.silu(h1) * h3)
    acc_ref[...] += jnp.dot(a, _bf(w2_ref[...]),
                            preferred_element_type=jnp.float32)

    @pl.when(j == NJ_S - 1)
    def _():
        o_ref[...] = acc_ref[...] + res_ref[...]


def _shared_expert_base(x2, sw1, sw3, sw2, hidden):
    bm = 256
    return pl.pallas_call(
        _shared_body,
        grid=(S // bm, NJ_S),
        in_specs=[
            pl.BlockSpec((bm, D), lambda i, j: (i, 0)),
            pl.BlockSpec((D, SH_BN), lambda i, j: (0, j)),
            pl.BlockSpec((D, SH_BN), lambda i, j: (0, j)),
            pl.BlockSpec((SH_BN, D), lambda i, j: (j, 0)),
            pl.BlockSpec((bm, D), lambda i, j: (i, 0)),
        ],
        out_specs=pl.BlockSpec((bm, D), lambda i, j: (i, 0)),
        out_shape=jax.ShapeDtypeStruct((S, D), jnp.float32),
        scratch_shapes=[pltpu.VMEM((bm, D), jnp.float32)],
        compiler_params=pltpu.CompilerParams(
            dimension_semantics=("arbitrary", "arbitrary")),
    )(x2, sw1, sw3, sw2, hidden)


def _gate_body(x_ref, gw_ref, o_ref):
    logits = jnp.dot(x_ref[...], gw_ref[...],
                     preferred_element_type=jnp.float32)
    iota = lax.broadcasted_iota(jnp.int32, logits.shape, 1)
    logits = jnp.where(iota < E, logits, -1e30)
    m = jnp.max(logits, axis=-1, keepdims=True)
    e = jnp.exp(logits - m)
    p = e / jnp.sum(e, axis=-1, keepdims=True)
    v0 = jnp.max(p, axis=-1, keepdims=True)
    i0 = jnp.min(jnp.where(p == v0, iota, 10 ** 9), axis=-1, keepdims=True)
    p2 = jnp.where(iota == i0, -1.0, p)
    v1 = jnp.max(p2, axis=-1, keepdims=True)
    i1 = jnp.min(jnp.where(p2 == v1, iota, 10 ** 9), axis=-1, keepdims=True)
    tot = v0 + v1
    cols = jnp.concatenate(
        [v0 / tot, v1 / tot, i0.astype(jnp.float32), i1.astype(jnp.float32),
         jnp.zeros((x_ref.shape[0], 124), jnp.float32)], axis=-1)
    o_ref[...] = cols


def _router(x2, gate_w):
    gw = jnp.pad(gate_w, ((0, 0), (0, 128 - E)))
    bm = 256
    out = pl.pallas_call(
        _gate_body,
        grid=(S // bm,),
        in_specs=[
            pl.BlockSpec((bm, D), lambda i: (i, 0)),
            pl.BlockSpec((D, 128), lambda i: (0, 0)),
        ],
        out_specs=pl.BlockSpec((bm, 128), lambda i: (i, 0)),
        out_shape=jax.ShapeDtypeStruct((S, 128), jnp.float32),
    )(x2, gw)
    v0, v1 = out[:, 0], out[:, 1]
    i0 = out[:, 2].astype(jnp.int32)
    i1 = out[:, 3].astype(jnp.int32)
    return v0, v1, i0, i1


def _group_body(m_ref, xg_ref, w1_ref, w3_ref, w2_ref, cw_ref, o_ref,
                acc_ref):
    i = pl.program_id(0)
    j = pl.program_id(1)
    alive = m_ref[NBLK + i] == 1

    @pl.when(jnp.logical_and(alive, j == 0))
    def _():
        acc_ref[...] = jnp.zeros_like(acc_ref)

    @pl.when(alive)
    def _():
        x = _bf(xg_ref[...])
        h1 = jnp.dot(x, _bf(w1_ref[0]), preferred_element_type=jnp.float32)
        h3 = jnp.dot(x, _bf(w3_ref[0]), preferred_element_type=jnp.float32)
        a = _bf(jax.nn.silu(h1) * h3)
        acc_ref[...] += jnp.dot(a, _bf(w2_ref[0]),
                                preferred_element_type=jnp.float32)

    @pl.when(jnp.logical_and(alive, j == NJ_E - 1))
    def _():
        o_ref[...] = acc_ref[...] * cw_ref[...]


def _grouped_ffn(block_meta, xg, ew1, ew3, ew2, cw):
    def _row(i, m):
        return jnp.where(m[NBLK + i] == 1, i, NBLK - 1)

    def _jj(i, j, m):
        return jnp.where(m[NBLK + i] == 1, j, 0)

    grid_spec = pltpu.PrefetchScalarGridSpec(
        num_scalar_prefetch=1,
        grid=(NBLK, NJ_E),
        in_specs=[
            pl.BlockSpec((BM, D), lambda i, j, m: (_row(i, m), 0)),
            pl.BlockSpec((1, D, GE_BN),
                         lambda i, j, m: (m[i], 0, _jj(i, j, m))),
            pl.BlockSpec((1, D, GE_BN),
                         lambda i, j, m: (m[i], 0, _jj(i, j, m))),
            pl.BlockSpec((1, GE_BN, D),
                         lambda i, j, m: (m[i], _jj(i, j, m), 0)),
            pl.BlockSpec((BM, 1), lambda i, j, m: (_row(i, m), 0)),
        ],
        out_specs=pl.BlockSpec((BM, D), lambda i, j, m: (_row(i, m), 0)),
        scratch_shapes=[pltpu.VMEM((BM, D), jnp.float32)],
    )
    return pl.pallas_call(
        _group_body,
        grid_spec=grid_spec,
        out_shape=jax.ShapeDtypeStruct((P, D), jnp.float32),
        compiler_params=pltpu.CompilerParams(
            dimension_semantics=("arbitrary", "arbitrary")),
    )(block_meta, xg, ew1, ew3, ew2, cw.reshape(P, 1))


def _sc_gather_rows(table, idx):
    n = idx.shape[0]
    d = table.shape[1]
    nw = 32
    chunk = 32
    per_w = n // nw
    iters = per_w // chunk
    mesh = plsc.VectorSubcoreMesh(core_axis_name="c", subcore_axis_name="s")

    @functools.partial(
        pl.kernel,
        mesh=mesh,
        out_type=jax.ShapeDtypeStruct((n, d), jnp.float32),
        scratch_types=[
            pltpu.VMEM((chunk,), jnp.int32),
            pltpu.VMEM((chunk, d), jnp.float32),
            pltpu.SemaphoreType.DMA,
        ],
    )
    def gk(table_hbm, idx_hbm, out_hbm, idx_v, rows_v, sem):
        wid = lax.axis_index("s") * 2 + lax.axis_index("c")
        base = wid * per_w

        def body(c, _):
            off = base + c * chunk
            pltpu.sync_copy(idx_hbm.at[pl.ds(off, chunk)], idx_v)
            pltpu.async_copy(table_hbm.at[idx_v], rows_v, sem).wait()
            pltpu.sync_copy(rows_v, out_hbm.at[pl.ds(off, chunk)])
            return ()

        lax.fori_loop(0, iters, body, ())

    return gk(table, idx)


def _combine_body(b_ref, g0_ref, g1_ref, o_ref):
    o_ref[...] = b_ref[...] + g0_ref[...] + g1_ref[...]


def _combine(base, g0, g1):
    bm = 256
    return pl.pallas_call(
        _combine_body,
        grid=(S // bm,),
        in_specs=[
            pl.BlockSpec((bm, D), lambda i: (i, 0)),
            pl.BlockSpec((bm, D), lambda i: (i, 0)),
            pl.BlockSpec((bm, D), lambda i: (i, 0)),
        ],
        out_specs=pl.BlockSpec((bm, D), lambda i: (i, 0)),
        out_shape=jax.ShapeDtypeStruct((S, D), jnp.float32),
    )(base, g0, g1)


def kernel(hidden_states, position_ids, ln1_g, ln1_b, ln2_g, ln2_b, wq, wk,
           wv, wkc, wvc, wqa, wqg, wov, wo, gate_w, ew1, ew2, ew3, sw1, sw2,
           sw3):
    h2d = hidden_states.reshape(S, D)

    xln = _layernorm(h2d, ln1_g, ln1_b)
    q = _matmul(xln, wq)
    k = _matmul(xln, wk)
    v = _matmul(xln, wv)

    inv_freq = 1.0 / (10000.0 ** (jnp.arange(0, ROPE, 2, jnp.float32) / ROPE))
    t = jnp.arange(4096, dtype=jnp.float32)
    freqs = jnp.outer(t, inv_freq)
    emb = jnp.concatenate((freqs, freqs), axis=-1)
    pos = position_ids.reshape(S)
    cos = jnp.cos(emb)[pos]
    sin = jnp.sin(emb)[pos]

    gated = _attention(q, k, v, cos, sin, wkc, wvc, wqa, wqg, wov)
    hidden = _matmul_add(gated, wo, h2d)

    x2 = _layernorm(hidden, ln2_g, ln2_b)
    base = _shared_expert_base(x2, sw1, sw3, sw2, hidden)

    v0, v1, i0, i1 = _router(x2, gate_w)

    e_f = jnp.stack([i0, i1], axis=1).reshape(-1)
    w_f = jnp.stack([v0, v1], axis=1).reshape(-1)
    oh = (e_f[:, None] == jnp.arange(E)[None, :]).astype(jnp.int32)
    ranks = jnp.take_along_axis(jnp.cumsum(oh, axis=0) - oh,
                                e_f[:, None], axis=1)[:, 0]
    counts = jnp.sum(oh, axis=0)
    padded = ((counts + BM - 1) // BM) * BM
    astart = jnp.concatenate([jnp.zeros(1, jnp.int32),
                              jnp.cumsum(padded)[:-1].astype(jnp.int32)])
    dest = astart[e_f] + ranks
    row_ids = (jnp.arange(P, dtype=jnp.int32) % S).at[dest].set(
        jnp.arange(S * TOPK, dtype=jnp.int32) // TOPK)
    cw = jnp.zeros(P, jnp.float32).at[dest].set(w_f)
    nblocks_e = padded // BM
    used = jnp.sum(nblocks_e).astype(jnp.int32)
    alive = (jnp.arange(NBLK, dtype=jnp.int32) < used).astype(jnp.int32)
    block_expert = jnp.repeat(jnp.arange(E, dtype=jnp.int32), nblocks_e,
                              total_repeat_length=NBLK)
    block_expert = jnp.where(alive == 1, block_expert,
                             jnp.take(block_expert, used - 1))
    block_meta = jnp.concatenate([block_expert, alive]).astype(jnp.int32)

    xg = _sc_gather_rows(x2, row_ids)
    outg = _grouped_ffn(block_meta, xg, ew1, ew3, ew2, cw)
    p01 = jnp.concatenate([dest[0::2], dest[1::2]])
    g01 = _sc_gather_rows(outg, p01)
    y = _combine(base, g01[:S], g01[S:])

    return y.reshape(1, S, D)


# speedup vs baseline: 1.6837x; 1.0029x over previous
import functools

import jax
import jax.numpy as jnp
from jax import lax
from jax.experimental import pallas as pl
from jax.experimental.pallas import tpu as pltpu
from jax.experimental.pallas import tpu_sc as plsc

S = 2048
D = 2048
H = 16
HD = 128
ROPE = 64
KVL = 64
FF = 4096
E = 8
TOPK = 2

BM = 512
P = S * TOPK + E * BM
NBLK = P // BM
GE_BN = 512
NJ_E = FF // GE_BN

SH_BN = 512
NJ_S = FF // SH_BN


def _ln_body(x_ref, g_ref, b_ref, o_ref):
    x = x_ref[...]
    m = jnp.mean(x, axis=-1, keepdims=True)
    v = jnp.mean((x - m) ** 2, axis=-1, keepdims=True)
    o_ref[...] = (x - m) * jax.lax.rsqrt(v + 1e-5) * g_ref[...] + b_ref[...]


def _layernorm(x, g, b):
    bm = 256
    return pl.pallas_call(
        _ln_body,
        grid=(S // bm,),
        in_specs=[
            pl.BlockSpec((bm, D), lambda i: (i, 0)),
            pl.BlockSpec((1, D), lambda i: (0, 0)),
            pl.BlockSpec((1, D), lambda i: (0, 0)),
        ],
        out_specs=pl.BlockSpec((bm, D), lambda i: (i, 0)),
        out_shape=jax.ShapeDtypeStruct((S, D), jnp.float32),
    )(x, g.reshape(1, D), b.reshape(1, D))


def _bf(x):
    return x.astype(jnp.bfloat16)


def _mm_body(x_ref, w_ref, o_ref):
    o_ref[...] = jnp.dot(_bf(x_ref[...]), _bf(w_ref[...]),
                         preferred_element_type=jnp.float32)


def _matmul(x, w):
    m, k = x.shape
    _, n = w.shape
    bm, bn = 256, 1024
    return pl.pallas_call(
        _mm_body,
        grid=(m // bm, n // bn),
        in_specs=[
            pl.BlockSpec((bm, k), lambda i, j: (i, 0)),
            pl.BlockSpec((k, bn), lambda i, j: (0, j)),
        ],
        out_specs=pl.BlockSpec((bm, bn), lambda i, j: (i, j)),
        out_shape=jax.ShapeDtypeStruct((m, n), jnp.float32),
    )(x, w)


def _mm_add_body(x_ref, w_ref, r_ref, o_ref):
    o_ref[...] = r_ref[...] + jnp.dot(_bf(x_ref[...]), _bf(w_ref[...]),
                                      preferred_element_type=jnp.float32)


def _matmul_add(x, w, res):
    m, k = x.shape
    _, n = w.shape
    bm, bn = 256, 1024
    return pl.pallas_call(
        _mm_add_body,
        grid=(m // bm, n // bn),
        in_specs=[
            pl.BlockSpec((bm, k), lambda i, j: (i, 0)),
            pl.BlockSpec((k, bn), lambda i, j: (0, j)),
            pl.BlockSpec((bm, bn), lambda i, j: (i, j)),
        ],
        out_specs=pl.BlockSpec((bm, bn), lambda i, j: (i, j)),
        out_shape=jax.ShapeDtypeStruct((m, n), jnp.float32),
    )(x, w, res)


def _rot_half(x):
    h = x.shape[-1] // 2
    return jnp.concatenate((-x[..., h:], x[..., :h]), axis=-1)


def _attn_body(q_ref, k_ref, v_ref, cos_ref, sin_ref, wkc_ref, wvc_ref,
               wqa_ref, wqg_ref, wov_ref, o_ref):
    cos = cos_ref[...]
    sin = sin_ref[...]
    q = q_ref[...]
    k = k_ref[...]
    qr = q[:, :ROPE] * cos + _rot_half(q[:, :ROPE]) * sin
    kr = k[:, :ROPE] * cos + _rot_half(k[:, :ROPE]) * sin
    qf = jnp.concatenate([qr, q[:, ROPE:]], axis=-1)
    kf = jnp.concatenate([kr, k[:, ROPE:]], axis=-1)
    k_c = jnp.dot(_bf(kf), _bf(wkc_ref[...]),
                  preferred_element_type=jnp.float32)
    v_c = jnp.dot(_bf(v_ref[...]), _bf(wvc_ref[...]),
                  preferred_element_type=jnp.float32)
    q_a = jnp.dot(_bf(qf), _bf(wqa_ref[...]),
                  preferred_element_type=jnp.float32)
    q_g = jnp.dot(_bf(qf), _bf(wqg_ref[...]),
                  preferred_element_type=jnp.float32)
    s = lax.dot_general(_bf(q_a), _bf(k_c), (((1,), (1,)), ((), ())),
                        preferred_element_type=jnp.float32)
    s = s * (1.0 / (KVL ** 0.5))
    s = s - jnp.max(s, axis=-1, keepdims=True)
    p = jnp.exp(s)
    p = p / jnp.sum(p, axis=-1, keepdims=True)
    o_c = jnp.dot(_bf(p), _bf(v_c), preferred_element_type=jnp.float32)
    up = jnp.dot(_bf(o_c), _bf(wov_ref[...]),
                 preferred_element_type=jnp.float32)
    o_ref[...] = jax.nn.silu(q_g) * up


def _attention(q, k, v, cos, sin, wkc, wvc, wqa, wqg, wov):
    return pl.pallas_call(
        _attn_body,
        grid=(H,),
        in_specs=[
            pl.BlockSpec((S, HD), lambda h: (0, h)),
            pl.BlockSpec((S, HD), lambda h: (0, h)),
            pl.BlockSpec((S, HD), lambda h: (0, h)),
            pl.BlockSpec((S, ROPE), lambda h: (0, 0)),
            pl.BlockSpec((S, ROPE), lambda h: (0, 0)),
            pl.BlockSpec((HD, KVL), lambda h: (0, 0)),
            pl.BlockSpec((HD, KVL), lambda h: (0, 0)),
            pl.BlockSpec((HD, KVL), lambda h: (0, 0)),
            pl.BlockSpec((HD, HD), lambda h: (0, 0)),
            pl.BlockSpec((KVL, HD), lambda h: (0, 0)),
        ],
        out_specs=pl.BlockSpec((S, HD), lambda h: (0, h)),
        out_shape=jax.ShapeDtypeStruct((S, D), jnp.float32),
    )(q, k, v, cos, sin, wkc, wvc, wqa, wqg, wov)


def _shared_body(x_ref, w1_ref, w3_ref, w2_ref, res_ref, o_ref, acc_ref):
    j = pl.program_id(1)

    @pl.when(j == 0)
    def _():
        acc_ref[...] = jnp.zeros_like(acc_ref)

    x = _bf(x_ref[...])
    h1 = jnp.dot(x, _bf(w1_ref[...]), preferred_element_type=jnp.float32)
    h3 = jnp.dot(x, _bf(w3_ref[...]), preferred_element_type=jnp.float32)
    a = _bf(jax.nn.silu(h1) * h3)
    acc_ref[...] += jnp.dot(a, _bf(w2_ref[...]),
                            preferred_element_type=jnp.float32)

    @pl.when(j == NJ_S - 1)
    def _():
        o_ref[...] = acc_ref[...] + res_ref[...]


def _shared_expert_base(x2, sw1, sw3, sw2, hidden):
    bm = 256
    return pl.pallas_call(
        _shared_body,
        grid=(S // bm, NJ_S),
        in_specs=[
            pl.BlockSpec((bm, D), lambda i, j: (i, 0)),
            pl.BlockSpec((D, SH_BN), lambda i, j: (0, j)),
            pl.BlockSpec((D, SH_BN), lambda i, j: (0, j)),
            pl.BlockSpec((SH_BN, D), lambda i, j: (j, 0)),
            pl.BlockSpec((bm, D), lambda i, j: (i, 0)),
        ],
        out_specs=pl.BlockSpec((bm, D), lambda i, j: (i, 0)),
        out_shape=jax.ShapeDtypeStruct((S, D), jnp.float32),
        scratch_shapes=[pltpu.VMEM((bm, D), jnp.float32)],
        compiler_params=pltpu.CompilerParams(
            dimension_semantics=("arbitrary", "arbitrary")),
    )(x2, sw1, sw3, sw2, hidden)


def _gate_body(x_ref, gw_ref, o_ref):
    logits = jnp.dot(x_ref[...], gw_ref[...],
                     preferred_element_type=jnp.float32)
    iota = lax.broadcasted_iota(jnp.int32, logits.shape, 1)
    logits = jnp.where(iota < E, logits, -1e30)
    m = jnp.max(logits, axis=-1, keepdims=True)
    e = jnp.exp(logits - m)
    p = e / jnp.sum(e, axis=-1, keepdims=True)
    v0 = jnp.max(p, axis=-1, keepdims=True)
    i0 = jnp.min(jnp.where(p == v0, iota, 10 ** 9), axis=-1, keepdims=True)
    p2 = jnp.where(iota == i0, -1.0, p)
    v1 = jnp.max(p2, axis=-1, keepdims=True)
    i1 = jnp.min(jnp.where(p2 == v1, iota, 10 ** 9), axis=-1, keepdims=True)
    tot = v0 + v1
    cols = jnp.concatenate(
        [v0 / tot, v1 / tot, i0.astype(jnp.float32), i1.astype(jnp.float32),
         jnp.zeros((x_ref.shape[0], 124), jnp.float32)], axis=-1)
    o_ref[...] = cols


def _router(x2, gate_w):
    gw = jnp.pad(gate_w, ((0, 0), (0, 128 - E)))
    bm = 256
    out = pl.pallas_call(
        _gate_body,
        grid=(S // bm,),
        in_specs=[
            pl.BlockSpec((bm, D), lambda i: (i, 0)),
            pl.BlockSpec((D, 128), lambda i: (0, 0)),
        ],
        out_specs=pl.BlockSpec((bm, 128), lambda i: (i, 0)),
        out_shape=jax.ShapeDtypeStruct((S, 128), jnp.float32),
    )(x2, gw)
    v0, v1 = out[:, 0], out[:, 1]
    i0 = out[:, 2].astype(jnp.int32)
    i1 = out[:, 3].astype(jnp.int32)
    return v0, v1, i0, i1


def _group_body(m_ref, xg_ref, w1_ref, w3_ref, w2_ref, cw_ref, o_ref,
                acc_ref):
    i = pl.program_id(0)
    j = pl.program_id(1)
    alive = m_ref[NBLK + i] == 1

    @pl.when(jnp.logical_and(alive, j == 0))
    def _():
        acc_ref[...] = jnp.zeros_like(acc_ref)

    @pl.when(alive)
    def _():
        x = _bf(xg_ref[...])
        h1 = jnp.dot(x, _bf(w1_ref[0]), preferred_element_type=jnp.float32)
        h3 = jnp.dot(x, _bf(w3_ref[0]), preferred_element_type=jnp.float32)
        a = _bf(jax.nn.silu(h1) * h3)
        acc_ref[...] += jnp.dot(a, _bf(w2_ref[0]),
                                preferred_element_type=jnp.float32)

    @pl.when(jnp.logical_and(alive, j == NJ_E - 1))
    def _():
        o_ref[...] = acc_ref[...] * cw_ref[...]


def _grouped_ffn(block_meta, xg, ew1, ew3, ew2, cw):
    def _row(i, m):
        return jnp.where(m[NBLK + i] == 1, i, NBLK - 1)

    def _jj(i, j, m):
        return jnp.where(m[NBLK + i] == 1, j, 0)

    grid_spec = pltpu.PrefetchScalarGridSpec(
        num_scalar_prefetch=1,
        grid=(NBLK, NJ_E),
        in_specs=[
            pl.BlockSpec((BM, D), lambda i, j, m: (_row(i, m), 0)),
            pl.BlockSpec((1, D, GE_BN),
                         lambda i, j, m: (m[i], 0, _jj(i, j, m))),
            pl.BlockSpec((1, D, GE_BN),
                         lambda i, j, m: (m[i], 0, _jj(i, j, m))),
            pl.BlockSpec((1, GE_BN, D),
                         lambda i, j, m: (m[i], _jj(i, j, m), 0)),
            pl.BlockSpec((BM, 1), lambda i, j, m: (_row(i, m), 0)),
        ],
        out_specs=pl.BlockSpec((BM, D), lambda i, j, m: (_row(i, m), 0)),
        scratch_shapes=[pltpu.VMEM((BM, D), jnp.float32)],
    )
    return pl.pallas_call(
        _group_body,
        grid_spec=grid_spec,
        out_shape=jax.ShapeDtypeStruct((P, D), jnp.float32),
        compiler_params=pltpu.CompilerParams(
            dimension_semantics=("arbitrary", "arbitrary")),
    )(block_meta, xg, ew1, ew3, ew2, cw.reshape(P, 1))


def _sc_gather_rows(table, idx):
    n = idx.shape[0]
    d = table.shape[1]
    nw = 32
    chunk = 32
    per_w = n // nw
    iters = per_w // chunk
    mesh = plsc.VectorSubcoreMesh(core_axis_name="c", subcore_axis_name="s")

    @functools.partial(
        pl.kernel,
        mesh=mesh,
        out_type=jax.ShapeDtypeStruct((n, d), jnp.float32),
        scratch_types=[
            pltpu.VMEM((chunk,), jnp.int32),
            pltpu.VMEM((chunk, d), jnp.float32),
            pltpu.SemaphoreType.DMA,
        ],
    )
    def gk(table_hbm, idx_hbm, out_hbm, idx_v, rows_v, sem):
        wid = lax.axis_index("s") * 2 + lax.axis_index("c")
        base = wid * per_w

        def body(c, _):
            off = base + c * chunk
            pltpu.sync_copy(idx_hbm.at[pl.ds(off, chunk)], idx_v)
            pltpu.async_copy(table_hbm.at[idx_v], rows_v, sem).wait()
            pltpu.sync_copy(rows_v, out_hbm.at[pl.ds(off, chunk)])
            return ()

        lax.fori_loop(0, iters, body, ())

    return gk(table, idx)


def _combine_body(b_ref, g0_ref, g1_ref, o_ref):
    o_ref[...] = b_ref[...] + g0_ref[...] + g1_ref[...]


def _combine(base, g0, g1):
    bm = 256
    return pl.pallas_call(
        _combine_body,
        grid=(S // bm,),
        in_specs=[
            pl.BlockSpec((bm, D), lambda i: (i, 0)),
            pl.BlockSpec((bm, D), lambda i: (i, 0)),
            pl.BlockSpec((bm, D), lambda i: (i, 0)),
        ],
        out_specs=pl.BlockSpec((bm, D), lambda i: (i, 0)),
        out_shape=jax.ShapeDtypeStruct((S, D), jnp.float32),
    )(base, g0, g1)


def kernel(hidden_states, position_ids, ln1_g, ln1_b, ln2_g, ln2_b, wq, wk,
           wv, wkc, wvc, wqa, wqg, wov, wo, gate_w, ew1, ew2, ew3, sw1, sw2,
           sw3):
    h2d = hidden_states.reshape(S, D)

    xln = _layernorm(h2d, ln1_g, ln1_b)
    q = _matmul(xln, wq)
    k = _matmul(xln, wk)
    v = _matmul(xln, wv)

    inv_freq = 1.0 / (10000.0 ** (jnp.arange(0, ROPE, 2, jnp.float32) / ROPE))
    t = jnp.arange(4096, dtype=jnp.float32)
    freqs = jnp.outer(t, inv_freq)
    emb = jnp.concatenate((freqs, freqs), axis=-1)
    pos = position_ids.reshape(S)
    cos = jnp.cos(emb)[pos]
    sin = jnp.sin(emb)[pos]

    gated = _attention(q, k, v, cos, sin, wkc, wvc, wqa, wqg, wov)
    hidden = _matmul_add(gated, wo, h2d)

    x2 = _layernorm(hidden, ln2_g, ln2_b)

    v0, v1, i0, i1 = _router(x2, gate_w)

    e_f = jnp.stack([i0, i1], axis=1).reshape(-1)
    w_f = jnp.stack([v0, v1], axis=1).reshape(-1)
    oh = (e_f[:, None] == jnp.arange(E)[None, :]).astype(jnp.int32)
    ranks = jnp.take_along_axis(jnp.cumsum(oh, axis=0) - oh,
                                e_f[:, None], axis=1)[:, 0]
    counts = jnp.sum(oh, axis=0)
    padded = ((counts + BM - 1) // BM) * BM
    astart = jnp.concatenate([jnp.zeros(1, jnp.int32),
                              jnp.cumsum(padded)[:-1].astype(jnp.int32)])
    dest = astart[e_f] + ranks
    row_ids = (jnp.arange(P, dtype=jnp.int32) % S).at[dest].set(
        jnp.arange(S * TOPK, dtype=jnp.int32) // TOPK)
    cw = jnp.zeros(P, jnp.float32).at[dest].set(w_f)
    nblocks_e = padded // BM
    used = jnp.sum(nblocks_e).astype(jnp.int32)
    alive = (jnp.arange(NBLK, dtype=jnp.int32) < used).astype(jnp.int32)
    block_expert = jnp.repeat(jnp.arange(E, dtype=jnp.int32), nblocks_e,
                              total_repeat_length=NBLK)
    block_expert = jnp.where(alive == 1, block_expert,
                             jnp.take(block_expert, used - 1))
    block_meta = jnp.concatenate([block_expert, alive]).astype(jnp.int32)

    xg = _sc_gather_rows(x2, row_ids)
    base = _shared_expert_base(x2, sw1, sw3, sw2, hidden)
    outg = _grouped_ffn(block_meta, xg, ew1, ew3, ew2, cw)
    p01 = jnp.concatenate([dest[0::2], dest[1::2]])
    g01 = _sc_gather_rows(outg, p01)
    y = _combine(base, g01[:S], g01[S:])

    return y.reshape(1, S, D)
